# baseline (speedup 1.0000x reference)
import jax
import jax.numpy as jnp
from jax.experimental import pallas as pl

_PACK = 4
_ROWS = 512


def _mlp_body(x_ref, w0_ref, b0_ref, w1_ref, b1_ref, o_ref):
    h = jnp.dot(x_ref[0], w0_ref[0], preferred_element_type=jnp.float32)
    h = h + jnp.dot(x_ref[1], w0_ref[1], preferred_element_type=jnp.float32)
    h = jnp.maximum(h + b0_ref[...], 0.0)
    o_ref[...] = jnp.dot(h, w1_ref[...], preferred_element_type=jnp.float32) + b1_ref[...]


def kernel(inputs, W0, b0, W1, b1):
    _, B, K = inputs.shape
    H = W0.shape[1]
    P = _PACK
    L = P * K
    HP = P * H
    NP = B // P

    xp = inputs.reshape(2, NP, L)

    w0_pack = jnp.zeros((2, L, HP), dtype=jnp.float32)
    w1_pack = jnp.zeros((HP, P), dtype=jnp.float32)
    for k in range(P):
        w0_pack = w0_pack.at[0, k * K:(k + 1) * K, k * H:(k + 1) * H].set(W0[:K])
        w0_pack = w0_pack.at[1, k * K:(k + 1) * K, k * H:(k + 1) * H].set(W0[K:])
        w1_pack = w1_pack.at[k * H:(k + 1) * H, k].set(W1[:, 0])
    b0_pack = jnp.tile(b0, (P,)).reshape(1, HP)
    b1_pack = jnp.broadcast_to(b1.reshape(1, 1), (1, P))

    out = pl.pallas_call(
        _mlp_body,
        grid=(NP // _ROWS,),
        in_specs=[
            pl.BlockSpec((2, _ROWS, L), lambda i: (0, i, 0)),
            pl.BlockSpec((2, L, HP), lambda i: (0, 0, 0)),
            pl.BlockSpec((1, HP), lambda i: (0, 0)),
            pl.BlockSpec((HP, P), lambda i: (0, 0)),
            pl.BlockSpec((1, P), lambda i: (0, 0)),
        ],
        out_specs=pl.BlockSpec((_ROWS, P), lambda i: (i, 0)),
        out_shape=jax.ShapeDtypeStruct((NP, P), jnp.float32),
    )(xp, w0_pack, b0_pack, w1_pack, b1_pack)
    return out.reshape(B, 1)
